# Initial kernel scaffold; baseline (speedup 1.0000x reference)
import jax
import jax.numpy as jnp
from jax.experimental import pallas as pl


def kernel(in_feat, edge_index, W1_self, W1_neigh, b1, W2_self, W2_neigh, b2):
    raise NotImplementedError("write your pallas kernel here")


# speedup vs baseline: 15.0842x; 15.0842x over previous
import functools

import jax
import jax.numpy as jnp
from jax import lax
from jax.experimental import pallas as pl
from jax.experimental.pallas import tpu as pltpu
from jax.experimental.pallas import tpu_sc as plsc

N_NODES = 50000
N_EDGES = 800000
H_FEATS = 150
F = 16
NW = 32
EB = 128
KCH = 196
EPAD = NW * KCH * EB
NPAD = 53248
RPS = NPAD // 16
BM = 2048
TC_GRID = 25


def _sc_edge_aggregate(table, srcb, dstb):
    mesh = plsc.VectorSubcoreMesh(core_axis_name="c", subcore_axis_name="s")

    @functools.partial(
        pl.kernel,
        mesh=mesh,
        out_type=jax.ShapeDtypeStruct((2, NPAD, F), jnp.float32),
        scratch_types=[
            pltpu.VMEM((KCH, EB), jnp.int32),
            pltpu.VMEM((KCH, EB), jnp.int32),
            pltpu.VMEM((EB, F), jnp.float32),
            pltpu.VMEM_SHARED((NPAD, F), jnp.float32),
            pltpu.SemaphoreType.DMA,
        ],
        compiler_params=pltpu.CompilerParams(use_tc_tiling_on_sc=False),
    )
    def k(table_h, src_h, dst_h, out_h, srcv, dstv, rows, aggs, sem):
        cid = lax.axis_index("c")
        sid = lax.axis_index("s")
        wid = sid * 2 + cid

        for i in range(EB):
            rows[i, :] = jnp.zeros((F,), jnp.float32)
        for z in range(RPS // EB):
            pltpu.sync_copy(rows, aggs.at[pl.ds(sid * RPS + z * EB, EB)])
        plsc.subcore_barrier()

        pltpu.sync_copy(src_h.at[wid], srcv)
        pltpu.sync_copy(dst_h.at[wid], dstv)

        def body(j, carry):
            pltpu.async_copy(table_h.at[srcv.at[j]], rows, sem).wait()
            pltpu.sync_copy(rows, aggs.at[dstv.at[j]], add=True)
            return carry

        lax.fori_loop(0, KCH, body, 0)
        plsc.subcore_barrier()

        pltpu.sync_copy(aggs.at[pl.ds(sid * RPS, RPS)],
                        out_h.at[cid, pl.ds(sid * RPS, RPS)])

    return k(table, srcb, dstb)


def _tc_layer1(xp, aggp, w1s, w1n, b1r, w2s, w2n, b2r):
    def body(xp_ref, agg_ref, w1s_ref, w1n_ref, b1_ref, w2s_ref, w2n_ref,
             b2_ref, p_ref, q_ref):
        a = agg_ref[0] + agg_ref[1]
        deg = jnp.clip(a[:, 3:4], 1.0, None)
        hn = a / deg
        h = (jnp.dot(xp_ref[...], w1s_ref[...], preferred_element_type=jnp.float32)
             + jnp.dot(hn, w1n_ref[...], preferred_element_type=jnp.float32)
             + b1_ref[...])
        h = jnp.maximum(h, 0.0)
        p_ref[...] = jnp.dot(h, w2n_ref[...], preferred_element_type=jnp.float32)
        q_ref[...] = (jnp.dot(h, w2s_ref[...], preferred_element_type=jnp.float32)
                      + b2_ref[...])

    return pl.pallas_call(
        body,
        grid=(TC_GRID,),
        in_specs=[
            pl.BlockSpec((BM, F), lambda i: (i, 0)),
            pl.BlockSpec((2, BM, F), lambda i: (0, i, 0)),
            pl.BlockSpec((F, H_FEATS), lambda i: (0, 0)),
            pl.BlockSpec((F, H_FEATS), lambda i: (0, 0)),
            pl.BlockSpec((1, H_FEATS), lambda i: (0, 0)),
            pl.BlockSpec((H_FEATS, F), lambda i: (0, 0)),
            pl.BlockSpec((H_FEATS, F), lambda i: (0, 0)),
            pl.BlockSpec((1, F), lambda i: (0, 0)),
        ],
        out_specs=[pl.BlockSpec((BM, F), lambda i: (i, 0)),
                   pl.BlockSpec((BM, F), lambda i: (i, 0))],
        out_shape=[jax.ShapeDtypeStruct((N_NODES, F), jnp.float32),
                   jax.ShapeDtypeStruct((N_NODES, F), jnp.float32)],
    )(xp, aggp, w1s, w1n, b1r, w2s, w2n, b2r)


def _tc_out(q, agg2p, agg1p):
    def body(q_ref, a2_ref, a1_ref, out_ref):
        a2 = a2_ref[0] + a2_ref[1]
        a1 = a1_ref[0] + a1_ref[1]
        deg = jnp.clip(a1[:, 3:4], 1.0, None)
        out_ref[...] = q_ref[...] + a2 / deg

    return pl.pallas_call(
        body,
        grid=(TC_GRID,),
        in_specs=[
            pl.BlockSpec((BM, F), lambda i: (i, 0)),
            pl.BlockSpec((2, BM, F), lambda i: (0, i, 0)),
            pl.BlockSpec((2, BM, F), lambda i: (0, i, 0)),
        ],
        out_specs=pl.BlockSpec((BM, F), lambda i: (i, 0)),
        out_shape=jax.ShapeDtypeStruct((N_NODES, F), jnp.float32),
    )(q, agg2p, agg1p)


def kernel(in_feat, edge_index, W1_self, W1_neigh, b1, W2_self, W2_neigh, b2):
    ei = edge_index.astype(jnp.int32)
    pad = EPAD - N_EDGES
    srcb = jnp.concatenate(
        [ei[0], jnp.zeros((pad,), jnp.int32)]).reshape(NW, KCH, EB)
    dstb = jnp.concatenate(
        [ei[1], jnp.full((pad,), N_NODES, jnp.int32)]).reshape(NW, KCH, EB)
    xp = jnp.concatenate(
        [in_feat,
         jnp.ones((N_NODES, 1), jnp.float32),
         jnp.zeros((N_NODES, F - 4), jnp.float32)], axis=1)
    w1s = jnp.zeros((F, H_FEATS), jnp.float32).at[0:3].set(W1_self)
    w1n = jnp.zeros((F, H_FEATS), jnp.float32).at[0:3].set(W1_neigh)

    agg1 = _sc_edge_aggregate(xp, srcb, dstb)
    p, q = _tc_layer1(xp, agg1, w1s, w1n, b1.reshape(1, H_FEATS),
                      W2_self, W2_neigh, b2.reshape(1, F))
    agg2 = _sc_edge_aggregate(p, srcb, dstb)
    return _tc_out(q, agg2, agg1)


# speedup vs baseline: 19.6263x; 1.3011x over previous
import functools

import jax
import jax.numpy as jnp
from jax import lax
from jax.experimental import pallas as pl
from jax.experimental.pallas import tpu as pltpu
from jax.experimental.pallas import tpu_sc as plsc

N_NODES = 50000
N_EDGES = 800000
H_FEATS = 150
F = 16
NW = 32
EB = 128
KCH = 196
EPAD = NW * KCH * EB
NPAD = 53248
RPS = NPAD // 16
BM = 2048
TC_GRID = 25


def _sc_edge_aggregate(table, srcb, dstb):
    mesh = plsc.VectorSubcoreMesh(core_axis_name="c", subcore_axis_name="s")

    @functools.partial(
        pl.kernel,
        mesh=mesh,
        out_type=jax.ShapeDtypeStruct((2, NPAD, F), jnp.float32),
        scratch_types=[
            pltpu.VMEM((KCH, EB), jnp.int32),
            pltpu.VMEM((KCH, EB), jnp.int32),
            pltpu.VMEM((EB, F), jnp.float32),
            pltpu.VMEM((EB, F), jnp.float32),
            pltpu.VMEM_SHARED((NPAD, F), jnp.float32),
            pltpu.SemaphoreType.DMA,
            pltpu.SemaphoreType.DMA,
        ],
        compiler_params=pltpu.CompilerParams(use_tc_tiling_on_sc=False),
    )
    def k(table_h, src_h, dst_h, out_h, srcv, dstv, rows0, rows1, aggs,
          sem0, sem1):
        cid = lax.axis_index("c")
        sid = lax.axis_index("s")
        wid = sid * 2 + cid

        for i in range(EB):
            rows0[i, :] = jnp.zeros((F,), jnp.float32)
        for z in range(RPS // EB):
            pltpu.sync_copy(rows0, aggs.at[pl.ds(sid * RPS + z * EB, EB)])
        plsc.subcore_barrier()

        pltpu.sync_copy(src_h.at[wid], srcv)
        pltpu.sync_copy(dst_h.at[wid], dstv)

        pltpu.async_copy(table_h.at[srcv.at[0]], rows0, sem0)

        def body(jh, carry):
            j = jh * 2
            pltpu.make_async_copy(table_h.at[srcv.at[j]], rows0, sem0).wait()
            pltpu.async_copy(table_h.at[srcv.at[j + 1]], rows1, sem1)
            pltpu.sync_copy(rows0, aggs.at[dstv.at[j]], add=True)

            @pl.when(j + 2 < KCH)
            def _():
                pltpu.async_copy(table_h.at[srcv.at[j + 2]], rows0, sem0)

            pltpu.make_async_copy(table_h.at[srcv.at[j + 1]], rows1, sem1).wait()
            pltpu.sync_copy(rows1, aggs.at[dstv.at[j + 1]], add=True)
            return carry

        lax.fori_loop(0, KCH // 2, body, 0)
        plsc.subcore_barrier()

        pltpu.sync_copy(aggs.at[pl.ds(sid * RPS, RPS)],
                        out_h.at[cid, pl.ds(sid * RPS, RPS)])

    return k(table, srcb, dstb)


def _tc_layer1(xp, aggp, w1s, w1n, b1r, w2s, w2n, b2r):
    def body(xp_ref, agg_ref, w1s_ref, w1n_ref, b1_ref, w2s_ref, w2n_ref,
             b2_ref, p_ref, q_ref):
        a = agg_ref[0] + agg_ref[1]
        deg = jnp.clip(a[:, 3:4], 1.0, None)
        hn = a / deg
        h = (jnp.dot(xp_ref[...], w1s_ref[...], preferred_element_type=jnp.float32)
             + jnp.dot(hn, w1n_ref[...], preferred_element_type=jnp.float32)
             + b1_ref[...])
        h = jnp.maximum(h, 0.0)
        p_ref[...] = jnp.dot(h, w2n_ref[...], preferred_element_type=jnp.float32)
        q_ref[...] = (jnp.dot(h, w2s_ref[...], preferred_element_type=jnp.float32)
                      + b2_ref[...])

    return pl.pallas_call(
        body,
        grid=(TC_GRID,),
        in_specs=[
            pl.BlockSpec((BM, F), lambda i: (i, 0)),
            pl.BlockSpec((2, BM, F), lambda i: (0, i, 0)),
            pl.BlockSpec((F, H_FEATS), lambda i: (0, 0)),
            pl.BlockSpec((F, H_FEATS), lambda i: (0, 0)),
            pl.BlockSpec((1, H_FEATS), lambda i: (0, 0)),
            pl.BlockSpec((H_FEATS, F), lambda i: (0, 0)),
            pl.BlockSpec((H_FEATS, F), lambda i: (0, 0)),
            pl.BlockSpec((1, F), lambda i: (0, 0)),
        ],
        out_specs=[pl.BlockSpec((BM, F), lambda i: (i, 0)),
                   pl.BlockSpec((BM, F), lambda i: (i, 0))],
        out_shape=[jax.ShapeDtypeStruct((N_NODES, F), jnp.float32),
                   jax.ShapeDtypeStruct((N_NODES, F), jnp.float32)],
    )(xp, aggp, w1s, w1n, b1r, w2s, w2n, b2r)


def _tc_out(q, agg2p, agg1p):
    def body(q_ref, a2_ref, a1_ref, out_ref):
        a2 = a2_ref[0] + a2_ref[1]
        a1 = a1_ref[0] + a1_ref[1]
        deg = jnp.clip(a1[:, 3:4], 1.0, None)
        out_ref[...] = q_ref[...] + a2 / deg

    return pl.pallas_call(
        body,
        grid=(TC_GRID,),
        in_specs=[
            pl.BlockSpec((BM, F), lambda i: (i, 0)),
            pl.BlockSpec((2, BM, F), lambda i: (0, i, 0)),
            pl.BlockSpec((2, BM, F), lambda i: (0, i, 0)),
        ],
        out_specs=pl.BlockSpec((BM, F), lambda i: (i, 0)),
        out_shape=jax.ShapeDtypeStruct((N_NODES, F), jnp.float32),
    )(q, agg2p, agg1p)


def kernel(in_feat, edge_index, W1_self, W1_neigh, b1, W2_self, W2_neigh, b2):
    ei = edge_index.astype(jnp.int32)
    pad = EPAD - N_EDGES
    srcb = jnp.concatenate(
        [ei[0], jnp.zeros((pad,), jnp.int32)]).reshape(NW, KCH, EB)
    dstb = jnp.concatenate(
        [ei[1], jnp.full((pad,), N_NODES, jnp.int32)]).reshape(NW, KCH, EB)
    xp = jnp.concatenate(
        [in_feat,
         jnp.ones((N_NODES, 1), jnp.float32),
         jnp.zeros((N_NODES, F - 4), jnp.float32)], axis=1)
    w1s = jnp.zeros((F, H_FEATS), jnp.float32).at[0:3].set(W1_self)
    w1n = jnp.zeros((F, H_FEATS), jnp.float32).at[0:3].set(W1_neigh)

    agg1 = _sc_edge_aggregate(xp, srcb, dstb)
    p, q = _tc_layer1(xp, agg1, w1s, w1n, b1.reshape(1, H_FEATS),
                      W2_self, W2_neigh, b2.reshape(1, F))
    agg2 = _sc_edge_aggregate(p, srcb, dstb)
    return _tc_out(q, agg2, agg1)


# speedup vs baseline: 19.6531x; 1.0014x over previous
import functools

import jax
import jax.numpy as jnp
from jax import lax
from jax.experimental import pallas as pl
from jax.experimental.pallas import tpu as pltpu
from jax.experimental.pallas import tpu_sc as plsc

N_NODES = 50000
N_EDGES = 800000
H_FEATS = 150
F = 16
NW = 32
EB = 128
KCH = 196
EPAD = NW * KCH * EB
NPAD = 53248
RPS = NPAD // 16
BM = 4096
TC_GRID = 13


def _sc_edge_aggregate(table, srcb, dstb, width):
    mesh = plsc.VectorSubcoreMesh(core_axis_name="c", subcore_axis_name="s")

    @functools.partial(
        pl.kernel,
        mesh=mesh,
        out_type=jax.ShapeDtypeStruct((2, NPAD, width), jnp.float32),
        scratch_types=[
            pltpu.VMEM((KCH, EB), jnp.int32),
            pltpu.VMEM((KCH, EB), jnp.int32),
            pltpu.VMEM((EB, width), jnp.float32),
            pltpu.VMEM((EB, width), jnp.float32),
            pltpu.VMEM_SHARED((NPAD, width), jnp.float32),
            pltpu.SemaphoreType.DMA,
            pltpu.SemaphoreType.DMA,
        ],
        compiler_params=pltpu.CompilerParams(use_tc_tiling_on_sc=False),
    )
    def k(table_h, src_h, dst_h, out_h, srcv, dstv, rows0, rows1, aggs,
          sem0, sem1):
        cid = lax.axis_index("c")
        sid = lax.axis_index("s")
        wid = sid * 2 + cid

        for i in range(EB):
            rows0[i, :] = jnp.zeros((width,), jnp.float32)
        for z in range(RPS // EB):
            pltpu.sync_copy(rows0, aggs.at[pl.ds(sid * RPS + z * EB, EB)])
        plsc.subcore_barrier()

        pltpu.sync_copy(src_h.at[wid], srcv)
        pltpu.sync_copy(dst_h.at[wid], dstv)

        pltpu.async_copy(table_h.at[srcv.at[0]], rows0, sem0)

        def body(jh, carry):
            j = jh * 2
            pltpu.make_async_copy(table_h.at[srcv.at[j]], rows0, sem0).wait()
            pltpu.async_copy(table_h.at[srcv.at[j + 1]], rows1, sem1)
            pltpu.sync_copy(rows0, aggs.at[dstv.at[j]], add=True)

            @pl.when(j + 2 < KCH)
            def _():
                pltpu.async_copy(table_h.at[srcv.at[j + 2]], rows0, sem0)

            pltpu.make_async_copy(table_h.at[srcv.at[j + 1]], rows1, sem1).wait()
            pltpu.sync_copy(rows1, aggs.at[dstv.at[j + 1]], add=True)
            return carry

        lax.fori_loop(0, KCH // 2, body, 0)
        plsc.subcore_barrier()

        pltpu.sync_copy(aggs.at[pl.ds(sid * RPS, RPS)],
                        out_h.at[cid, pl.ds(sid * RPS, RPS)])

    return k(table, srcb, dstb)


def _tc_layer1(x, aggp, w1s, w1n4, b1r, w2s, w2n, b2r):
    def body(x_ref, agg_ref, w1s_ref, w1n_ref, b1_ref, w2s_ref, w2n_ref,
             b2_ref, p_ref, q_ref):
        a = agg_ref[0] + agg_ref[1]
        deg = jnp.clip(a[:, 3:4], 1.0, None)
        hn = a / deg
        h = (jnp.dot(x_ref[...], w1s_ref[...], preferred_element_type=jnp.float32)
             + jnp.dot(hn, w1n_ref[...], preferred_element_type=jnp.float32)
             + b1_ref[...])
        h = jnp.maximum(h, 0.0)
        p_ref[...] = jnp.dot(h, w2n_ref[...], preferred_element_type=jnp.float32)
        q_ref[...] = (jnp.dot(h, w2s_ref[...], preferred_element_type=jnp.float32)
                      + b2_ref[...])

    return pl.pallas_call(
        body,
        grid=(TC_GRID,),
        in_specs=[
            pl.BlockSpec((BM, 3), lambda i: (i, 0)),
            pl.BlockSpec((2, BM, F), lambda i: (0, i, 0)),
            pl.BlockSpec((3, H_FEATS), lambda i: (0, 0)),
            pl.BlockSpec((F, H_FEATS), lambda i: (0, 0)),
            pl.BlockSpec((1, H_FEATS), lambda i: (0, 0)),
            pl.BlockSpec((H_FEATS, F), lambda i: (0, 0)),
            pl.BlockSpec((H_FEATS, F), lambda i: (0, 0)),
            pl.BlockSpec((1, F), lambda i: (0, 0)),
        ],
        out_specs=[pl.BlockSpec((BM, F), lambda i: (i, 0)),
                   pl.BlockSpec((BM, F), lambda i: (i, 0))],
        out_shape=[jax.ShapeDtypeStruct((N_NODES, F), jnp.float32),
                   jax.ShapeDtypeStruct((N_NODES, F), jnp.float32)],
    )(x, aggp, w1s, w1n4, b1r, w2s, w2n, b2r)


def _tc_out(q, agg2p, agg1p):
    def body(q_ref, a2_ref, a1_ref, out_ref):
        a2 = a2_ref[0] + a2_ref[1]
        a1 = a1_ref[0] + a1_ref[1]
        deg = jnp.clip(a1[:, 3:4], 1.0, None)
        out_ref[...] = q_ref[...] + a2 / deg

    return pl.pallas_call(
        body,
        grid=(TC_GRID,),
        in_specs=[
            pl.BlockSpec((BM, F), lambda i: (i, 0)),
            pl.BlockSpec((2, BM, F), lambda i: (0, i, 0)),
            pl.BlockSpec((2, BM, F), lambda i: (0, i, 0)),
        ],
        out_specs=pl.BlockSpec((BM, F), lambda i: (i, 0)),
        out_shape=jax.ShapeDtypeStruct((N_NODES, F), jnp.float32),
    )(q, agg2p, agg1p)


def kernel(in_feat, edge_index, W1_self, W1_neigh, b1, W2_self, W2_neigh, b2):
    ei = edge_index.astype(jnp.int32)
    pad = EPAD - N_EDGES
    srcb = jnp.concatenate(
        [ei[0], jnp.zeros((pad,), jnp.int32)]).reshape(NW, KCH, EB)
    dstb = jnp.concatenate(
        [ei[1], jnp.full((pad,), N_NODES, jnp.int32)]).reshape(NW, KCH, EB)
    xp = jnp.concatenate(
        [in_feat, jnp.ones((N_NODES, 1), jnp.float32),
         jnp.zeros((N_NODES, F - 4), jnp.float32)], axis=1)
    w1n4 = jnp.concatenate(
        [W1_neigh, jnp.zeros((F - 3, H_FEATS), jnp.float32)], axis=0)

    agg1 = _sc_edge_aggregate(xp, srcb, dstb, F)
    p, q = _tc_layer1(in_feat, agg1, W1_self, w1n4, b1.reshape(1, H_FEATS),
                      W2_self, W2_neigh, b2.reshape(1, F))
    agg2 = _sc_edge_aggregate(p, srcb, dstb, F)
    return _tc_out(q, agg2, agg1)
